# baseline (speedup 1.0000x reference)
import functools

import jax
import jax.numpy as jnp
from jax import lax
from jax.experimental import pallas as pl
from jax.experimental.pallas import tpu as pltpu
from jax.experimental.pallas import tpu_sc as plsc

K_KEEP = 512
B_SZ = 2
L_SEQ = 2048
D_MODEL = 768
H_HEADS = 12
QC = 2048


def _score_body(s_ref, out_ref, acc_ref, comp_ref):
    h = pl.program_id(1)
    qi = pl.program_id(2)

    @pl.when(jnp.logical_and(h == 0, qi == 0))
    def _per_batch_init():
        acc_ref[...] = jnp.zeros_like(acc_ref)
        comp_ref[...] = jnp.zeros_like(comp_ref)

    chunk = s_ref[0, 0]
    part = jnp.sum(chunk.reshape(8, QC // 8, L_SEQ), axis=1)
    csum = jnp.sum(part, axis=0)[None, :]
    y = csum - comp_ref[...]
    t = acc_ref[...] + y
    comp_ref[...] = (t - acc_ref[...]) - y
    acc_ref[...] = t

    @pl.when(jnp.logical_and(h == H_HEADS - 1, qi == L_SEQ // QC - 1))
    def _per_batch_fini():
        out_ref[0] = acc_ref[...] / jnp.float32(H_HEADS)


def _scores(self_attention_scores):
    grid = (B_SZ, H_HEADS, L_SEQ // QC)
    return pl.pallas_call(
        _score_body,
        grid=grid,
        in_specs=[
            pl.BlockSpec((1, 1, QC, L_SEQ), lambda b, h, q: (b, h, q, 0)),
        ],
        out_specs=pl.BlockSpec((1, 1, L_SEQ), lambda b, h, q: (b, 0, 0)),
        out_shape=jax.ShapeDtypeStruct((B_SZ, 1, L_SEQ), jnp.float32),
        scratch_shapes=[
            pltpu.VMEM((1, L_SEQ), jnp.float32),
            pltpu.VMEM((1, L_SEQ), jnp.float32),
        ],
    )(self_attention_scores)


def _prefix_incl(x):
    n = x.shape[-1]
    k = 1
    while k < n:
        shifted = jnp.concatenate(
            [jnp.zeros(x.shape[:-1] + (k,), x.dtype), x[..., :-k]], axis=-1)
        x = x + shifted
        k *= 2
    return x


LC = 256


def _select_body(scores_ref, am_ref, hid_ref, idx_ref, pam_ref, new_ref,
                 sel_scr, acc_scr):
    i = pl.program_id(0)

    @pl.when(i == 0)
    def _selection():
        _do_selection(scores_ref, am_ref, idx_ref, pam_ref, sel_scr)
        acc_scr[...] = jnp.zeros_like(acc_scr)

    selc = sel_scr[:, pl.ds(i * LC, LC)]
    for b in range(B_SZ):
        part = jnp.sum(hid_ref[b] * (1.0 - selc[b])[:, None], axis=0)
        acc_scr[pl.ds(b, 1), :] += part[None, :]

    @pl.when(i == L_SEQ // LC - 1)
    def _fini():
        new_ref[...] = acc_scr[...] / jnp.float32(L_SEQ - K_KEEP)


def _do_selection(scores_ref, am_ref, idx_ref, pam_ref, sel_scr):
    s = scores_ref[:, 0, :]
    bits = lax.bitcast_convert_type(s, jnp.int32)
    key = jnp.where(bits >= 0, bits, -(bits & jnp.int32(0x7FFFFFFF)))

    def step(_, carry):
        lo, hi = carry
        mid = (lo >> 1) + (hi >> 1) + (lo & hi & 1) + ((lo ^ hi) & 1)
        cnt = jnp.sum((key >= mid).astype(jnp.int32), axis=1, keepdims=True)
        take = cnt >= K_KEEP
        return (jnp.where(take, mid, lo), jnp.where(take, hi, mid - 1))

    lo0 = jnp.full((B_SZ, 1), jnp.int32(-2147483648))
    hi0 = jnp.full((B_SZ, 1), jnp.int32(2147483647))
    thr, _ = lax.fori_loop(0, 32, step, (lo0, hi0))

    gt = key > thr
    eq = key == thr
    n_gt = jnp.sum(gt.astype(jnp.int32), axis=1, keepdims=True)
    quota = K_KEEP - n_gt
    eqi = eq.astype(jnp.int32)
    eqb = _prefix_incl(eqi) - eqi
    sel = gt | (eq & (eqb < quota))
    self_f = sel.astype(jnp.float32)
    pos = _prefix_incl(self_f) - self_f
    sel_valid = jnp.where(sel, pos, jnp.float32(2 * L_SEQ))

    jdxf = lax.broadcasted_iota(jnp.int32, (1, L_SEQ), 1).astype(jnp.float32)
    KC = 128
    for b in range(B_SZ):
        sv_row = sel_valid[b:b + 1, :]
        am_row = am_ref[pl.ds(b, 1), :]
        for c in range(K_KEEP // KC):
            kdx = (lax.broadcasted_iota(jnp.int32, (KC, 1), 0) + c * KC
                   ).astype(jnp.float32)
            onehot = (sv_row == kdx).astype(jnp.float32)
            idxs = jnp.sum(onehot * jdxf, axis=1)
            pams = jnp.sum(onehot * am_row, axis=1)
            gl = idxs + jnp.float32(b * L_SEQ)
            idx_ref[b, pl.ds(c * KC, KC), :] = gl.astype(jnp.int32)[:, None]
            pam_ref[b, pl.ds(c * KC, KC), :] = pams[:, None]
    sel_scr[...] = self_f


def _select(scores, am_row, hidden_states):
    grid = (L_SEQ // LC,)
    return pl.pallas_call(
        _select_body,
        grid=grid,
        in_specs=[
            pl.BlockSpec((B_SZ, 1, L_SEQ), lambda i: (0, 0, 0)),
            pl.BlockSpec((B_SZ, L_SEQ), lambda i: (0, 0)),
            pl.BlockSpec((B_SZ, LC, D_MODEL), lambda i: (0, i, 0)),
        ],
        out_specs=[
            pl.BlockSpec((B_SZ, K_KEEP, 1), lambda i: (0, 0, 0)),
            pl.BlockSpec((B_SZ, K_KEEP, 1), lambda i: (0, 0, 0)),
            pl.BlockSpec((B_SZ, D_MODEL), lambda i: (0, 0)),
        ],
        out_shape=[
            jax.ShapeDtypeStruct((B_SZ, K_KEEP, 1), jnp.int32),
            jax.ShapeDtypeStruct((B_SZ, K_KEEP, 1), jnp.float32),
            jax.ShapeDtypeStruct((B_SZ, D_MODEL), jnp.float32),
        ],
        scratch_shapes=[
            pltpu.VMEM((B_SZ, L_SEQ), jnp.float32),
            pltpu.VMEM((B_SZ, D_MODEL), jnp.float32),
        ],
    )(scores, am_row, hidden_states)


@functools.lru_cache(maxsize=1)
def _make_sc_gather():
    info = plsc.get_sparse_core_info()
    nw = info.num_cores * info.num_subcores
    rows_total = B_SZ * K_KEEP
    rpw = rows_total // nw

    mesh = plsc.VectorSubcoreMesh(core_axis_name="c", subcore_axis_name="s")

    @functools.partial(
        pl.kernel,
        out_type=jax.ShapeDtypeStruct((rows_total, D_MODEL), jnp.float32),
        mesh=mesh,
        scratch_types=[
            pltpu.VMEM((rpw,), jnp.int32),
            pltpu.VMEM((rpw, D_MODEL), jnp.float32),
            pltpu.SemaphoreType.DMA,
        ],
    )
    def gather(table_hbm, idx_hbm, out_hbm, idx_v, rows_v, sem):
        wid = lax.axis_index("s") * info.num_cores + lax.axis_index("c")
        base = wid * rpw
        pltpu.sync_copy(idx_hbm.at[pl.ds(base, rpw)], idx_v)
        pltpu.async_copy(table_hbm.at[idx_v], rows_v, sem).wait()
        pltpu.sync_copy(rows_v, out_hbm.at[pl.ds(base, rpw)])

    return gather


def kernel(hidden_states, attention_mask, self_attention_scores):
    scores = _scores(self_attention_scores)
    am_row = attention_mask[:, 0, 0, :]
    idx3, pam3, new_token = _select(scores, am_row, hidden_states)

    gidx = idx3.reshape(-1)
    table = hidden_states.reshape(B_SZ * L_SEQ, D_MODEL)
    preserved_flat = _make_sc_gather()(table, gidx)
    preserved = preserved_flat.reshape(B_SZ, K_KEEP, D_MODEL)

    final_token = jnp.concatenate(
        [hidden_states[:, :1, :], preserved, new_token[:, None, :]], axis=1)
    zero = jnp.zeros((B_SZ, 1), attention_mask.dtype)
    final_attention_mask = jnp.concatenate(
        [zero, pam3[:, :, 0], zero], axis=-1)[:, None, None, :]
    return (final_token, final_attention_mask)


# speedup vs baseline: 1.0129x; 1.0129x over previous
import functools

import jax
import jax.numpy as jnp
from jax import lax
from jax.experimental import pallas as pl
from jax.experimental.pallas import tpu as pltpu
from jax.experimental.pallas import tpu_sc as plsc

K_KEEP = 512
B_SZ = 2
L_SEQ = 2048
D_MODEL = 768
H_HEADS = 12
QC = 2048


def _score_body(s_ref, out_ref, acc_ref, comp_ref):
    h = pl.program_id(1)
    qi = pl.program_id(2)

    @pl.when(jnp.logical_and(h == 0, qi == 0))
    def _per_batch_init():
        acc_ref[...] = jnp.zeros_like(acc_ref)
        comp_ref[...] = jnp.zeros_like(comp_ref)

    chunk = s_ref[0, 0]
    part = jnp.sum(chunk.reshape(8, QC // 8, L_SEQ), axis=1)
    csum = jnp.sum(part, axis=0)[None, :]
    y = csum - comp_ref[...]
    t = acc_ref[...] + y
    comp_ref[...] = (t - acc_ref[...]) - y
    acc_ref[...] = t

    @pl.when(jnp.logical_and(h == H_HEADS - 1, qi == L_SEQ // QC - 1))
    def _per_batch_fini():
        out_ref[0] = acc_ref[...] / jnp.float32(H_HEADS)


def _scores(self_attention_scores):
    grid = (B_SZ, H_HEADS, L_SEQ // QC)
    return pl.pallas_call(
        _score_body,
        grid=grid,
        in_specs=[
            pl.BlockSpec((1, 1, QC, L_SEQ), lambda b, h, q: (b, h, q, 0)),
        ],
        out_specs=pl.BlockSpec((1, 1, L_SEQ), lambda b, h, q: (b, 0, 0)),
        out_shape=jax.ShapeDtypeStruct((B_SZ, 1, L_SEQ), jnp.float32),
        scratch_shapes=[
            pltpu.VMEM((1, L_SEQ), jnp.float32),
            pltpu.VMEM((1, L_SEQ), jnp.float32),
        ],
    )(self_attention_scores)


def _prefix_incl(x):
    n = x.shape[-1]
    k = 1
    while k < n:
        shifted = jnp.concatenate(
            [jnp.zeros(x.shape[:-1] + (k,), x.dtype), x[..., :-k]], axis=-1)
        x = x + shifted
        k *= 2
    return x


def _select_body(scores_ref, am_ref, hid_ref, idx_ref, pam_ref, new_ref):
    s = scores_ref[:, 0, :]
    bits = lax.bitcast_convert_type(s, jnp.int32)
    key = jnp.where(bits >= 0, bits, -(bits & jnp.int32(0x7FFFFFFF)))

    def step(_, carry):
        lo, hi = carry
        mid = (lo >> 1) + (hi >> 1) + (lo & hi & 1) + ((lo ^ hi) & 1)
        cnt = jnp.sum((key >= mid).astype(jnp.int32), axis=1, keepdims=True)
        take = cnt >= K_KEEP
        return (jnp.where(take, mid, lo), jnp.where(take, hi, mid - 1))

    lo0 = jnp.full((B_SZ, 1), jnp.int32(-2147483648))
    hi0 = jnp.full((B_SZ, 1), jnp.int32(2147483647))
    thr, _ = lax.fori_loop(0, 32, step, (lo0, hi0))

    gt = key > thr
    eq = key == thr
    n_gt = jnp.sum(gt.astype(jnp.int32), axis=1, keepdims=True)
    quota = K_KEEP - n_gt
    eqi = eq.astype(jnp.int32)
    eqb = _prefix_incl(eqi) - eqi
    sel = gt | (eq & (eqb < quota))
    self_f = sel.astype(jnp.float32)
    pos = _prefix_incl(self_f) - self_f
    sel_valid = jnp.where(sel, pos, jnp.float32(2 * L_SEQ))

    jdxf = lax.broadcasted_iota(jnp.int32, (1, L_SEQ), 1).astype(jnp.float32)
    KC = 128
    for b in range(B_SZ):
        sv_row = sel_valid[b:b + 1, :]
        am_row = am_ref[pl.ds(b, 1), :]
        for c in range(K_KEEP // KC):
            kdx = (lax.broadcasted_iota(jnp.int32, (KC, 1), 0) + c * KC
                   ).astype(jnp.float32)
            onehot = (sv_row == kdx).astype(jnp.float32)
            idxs = jnp.sum(onehot * jdxf, axis=1)
            pams = jnp.sum(onehot * am_row, axis=1)
            gl = idxs + jnp.float32(b * L_SEQ)
            idx_ref[b, pl.ds(c * KC, KC), :] = gl.astype(jnp.int32)[:, None]
            pam_ref[b, pl.ds(c * KC, KC), :] = pams[:, None]
        unimp = jnp.sum(hid_ref[b] * (1.0 - self_f[b])[:, None], axis=0)
        new_ref[pl.ds(b, 1), :] = unimp[None, :] / jnp.float32(L_SEQ - K_KEEP)


def _select(scores, am_row, hidden_states):
    return pl.pallas_call(
        _select_body,
        in_specs=[
            pl.BlockSpec(memory_space=pltpu.VMEM),
            pl.BlockSpec(memory_space=pltpu.VMEM),
            pl.BlockSpec(memory_space=pltpu.VMEM),
        ],
        out_specs=[
            pl.BlockSpec(memory_space=pltpu.VMEM),
            pl.BlockSpec(memory_space=pltpu.VMEM),
            pl.BlockSpec(memory_space=pltpu.VMEM),
        ],
        out_shape=[
            jax.ShapeDtypeStruct((B_SZ, K_KEEP, 1), jnp.int32),
            jax.ShapeDtypeStruct((B_SZ, K_KEEP, 1), jnp.float32),
            jax.ShapeDtypeStruct((B_SZ, D_MODEL), jnp.float32),
        ],
    )(scores, am_row, hidden_states)


@functools.lru_cache(maxsize=1)
def _make_sc_gather():
    info = plsc.get_sparse_core_info()
    nw = info.num_cores * info.num_subcores
    rows_total = B_SZ * K_KEEP
    rpw = rows_total // nw

    mesh = plsc.VectorSubcoreMesh(core_axis_name="c", subcore_axis_name="s")

    @functools.partial(
        pl.kernel,
        out_type=jax.ShapeDtypeStruct((rows_total, D_MODEL), jnp.float32),
        mesh=mesh,
        scratch_types=[
            pltpu.VMEM((rpw,), jnp.int32),
            pltpu.VMEM((rpw, D_MODEL), jnp.float32),
            pltpu.SemaphoreType.DMA,
        ],
    )
    def gather(table_hbm, idx_hbm, out_hbm, idx_v, rows_v, sem):
        wid = lax.axis_index("s") * info.num_cores + lax.axis_index("c")
        base = wid * rpw
        pltpu.sync_copy(idx_hbm.at[pl.ds(base, rpw)], idx_v)
        pltpu.async_copy(table_hbm.at[idx_v], rows_v, sem).wait()
        pltpu.sync_copy(rows_v, out_hbm.at[pl.ds(base, rpw)])

    return gather


def kernel(hidden_states, attention_mask, self_attention_scores):
    scores = _scores(self_attention_scores)
    am_row = attention_mask[:, 0, 0, :]
    idx3, pam3, new_token = _select(scores, am_row, hidden_states)

    gidx = idx3.reshape(-1)
    table = hidden_states.reshape(B_SZ * L_SEQ, D_MODEL)
    preserved_flat = _make_sc_gather()(table, gidx)
    preserved = preserved_flat.reshape(B_SZ, K_KEEP, D_MODEL)

    final_token = jnp.concatenate(
        [hidden_states[:, :1, :], preserved, new_token[:, None, :]], axis=1)
    zero = jnp.zeros((B_SZ, 1), attention_mask.dtype)
    final_attention_mask = jnp.concatenate(
        [zero, pam3[:, :, 0], zero], axis=-1)[:, None, None, :]
    return (final_token, final_attention_mask)
